# Initial kernel scaffold; baseline (speedup 1.0000x reference)
import jax
import jax.numpy as jnp
from jax.experimental import pallas as pl


def kernel(inputs, targets):
    raise NotImplementedError("write your pallas kernel here")


# speedup vs baseline: 1.1120x; 1.1120x over previous
import jax
import jax.numpy as jnp
from jax.experimental import pallas as pl
from jax.experimental.pallas import tpu as pltpu

N = 8192
BLK = 512
NB = N // BLK


def _chamfer_body(x_ref, y_ref, out_ref, ymin_ref, xsum_ref):
    i = pl.program_id(0)

    @pl.when(i == 0)
    def _init():
        ymin_ref[...] = jnp.full((1, N), jnp.inf, dtype=jnp.float32)
        xsum_ref[0] = 0.0

    d = jnp.abs(x_ref[...] - y_ref[...])
    xmin = jnp.min(d, axis=1)
    xsum_ref[0] += jnp.sum(xmin)
    ymin_ref[...] = jnp.minimum(ymin_ref[...], jnp.min(d, axis=0, keepdims=True))

    @pl.when(i == NB - 1)
    def _fin():
        ysum = jnp.sum(ymin_ref[...])
        loss = (0.5 / N) * xsum_ref[0] + (0.5 / N) * ysum
        out_ref[...] = jnp.full((1, 1), loss, dtype=jnp.float32)


def kernel(inputs, targets):
    x = inputs.reshape(N, 1)
    y = targets.reshape(1, N)
    out = pl.pallas_call(
        _chamfer_body,
        grid=(NB,),
        in_specs=[
            pl.BlockSpec((BLK, 1), lambda i: (i, 0)),
            pl.BlockSpec((1, N), lambda i: (0, 0)),
        ],
        out_specs=pl.BlockSpec((1, 1), lambda i: (0, 0)),
        out_shape=jax.ShapeDtypeStruct((1, 1), jnp.float32),
        scratch_shapes=[
            pltpu.VMEM((1, N), jnp.float32),
            pltpu.SMEM((1,), jnp.float32),
        ],
    )(x, y)
    return out[0, 0]


# speedup vs baseline: 6.3976x; 5.7532x over previous
import jax
import jax.numpy as jnp
from jax import lax
from jax.experimental import pallas as pl

N = 8192
M = 2 * N
R = 128
C = 128
LOG_M = 14
NEG = float("-inf")
POS = float("inf")


def _xor_shuffle(z, j, r_iota, c_iota):
    if j < C:
        fwd = jnp.roll(z, -j, axis=1)
        bwd = jnp.roll(z, j, axis=1)
        return jnp.where((c_iota & j) == 0, fwd, bwd)
    jr = j // C
    fwd = jnp.roll(z, -jr, axis=0)
    bwd = jnp.roll(z, jr, axis=0)
    return jnp.where((r_iota & jr) == 0, fwd, bwd)


def _lane_scan(v, c_iota, op, fill):
    for s in (1, 2, 4, 8, 16, 32, 64):
        sh = jnp.roll(v, s, axis=1)
        sh = jnp.where(c_iota >= s, sh, fill)
        v = op(v, sh)
    return v


def _lane_scan_rev(v, c_iota, op, fill):
    for s in (1, 2, 4, 8, 16, 32, 64):
        sh = jnp.roll(v, -s, axis=1)
        sh = jnp.where(c_iota < C - s, sh, fill)
        v = op(v, sh)
    return v


def _row_scan_excl(col, r_iota1, op, fill):
    e = jnp.roll(col, 1, axis=0)
    e = jnp.where(r_iota1 >= 1, e, fill)
    for s in (1, 2, 4, 8, 16, 32, 64):
        sh = jnp.roll(e, s, axis=0)
        sh = jnp.where(r_iota1 >= s, sh, fill)
        e = op(e, sh)
    return e


def _row_scan_excl_rev(col, r_iota1, op, fill):
    e = jnp.roll(col, -1, axis=0)
    e = jnp.where(r_iota1 < R - 1, e, fill)
    for s in (1, 2, 4, 8, 16, 32, 64):
        sh = jnp.roll(e, -s, axis=0)
        sh = jnp.where(r_iota1 < R - s, sh, fill)
        e = op(e, sh)
    return e


def _cummax_incl(a, r_iota1, c_iota):
    v = _lane_scan(a, c_iota, jnp.maximum, NEG)
    rm = lax.slice(v, (0, C - 1), (R, C))
    e = _row_scan_excl(rm, r_iota1, jnp.maximum, NEG)
    return jnp.maximum(v, e)


def _revcummin_incl(a, r_iota1, c_iota):
    v = _lane_scan_rev(a, c_iota, jnp.minimum, POS)
    rm = lax.slice(v, (0, 0), (R, 1))
    e = _row_scan_excl_rev(rm, r_iota1, jnp.minimum, POS)
    return jnp.minimum(v, e)


def _chamfer_body(z_ref, out_ref):
    z = z_ref[...]
    r_iota = lax.broadcasted_iota(jnp.int32, (R, C), 0)
    c_iota = lax.broadcasted_iota(jnp.int32, (R, C), 1)
    r_iota1 = lax.broadcasted_iota(jnp.int32, (R, 1), 0)
    p = r_iota * C + c_iota

    zi = lax.bitcast_convert_type(z, jnp.int32)
    zi = jnp.where(p >= N, zi | 1, zi & jnp.int32(~1))
    z = lax.bitcast_convert_type(zi, jnp.float32)

    for kk in range(1, LOG_M + 1):
        kbit = 1 << kk
        for jj in range(kk - 1, -1, -1):
            j = 1 << jj
            partner = _xor_shuffle(z, j, r_iota, c_iota)
            wantmin = ((p & j) == 0) == ((p & kbit) == 0)
            mn = jnp.minimum(z, partner)
            mx = jnp.maximum(z, partner)
            z = jnp.where(wantmin, mn, mx)

    zi2 = lax.bitcast_convert_type(z, jnp.int32)
    is_y = (zi2 & 1) == 1

    ly = _cummax_incl(jnp.where(is_y, z, NEG), r_iota1, c_iota)
    ry = _revcummin_incl(jnp.where(is_y, z, POS), r_iota1, c_iota)
    dx = jnp.minimum(z - ly, ry - z)
    sum_x = jnp.sum(jnp.where(is_y, 0.0, dx))

    lx = _cummax_incl(jnp.where(is_y, NEG, z), r_iota1, c_iota)
    rx = _revcummin_incl(jnp.where(is_y, POS, z), r_iota1, c_iota)
    dy = jnp.minimum(z - lx, rx - z)
    sum_y = jnp.sum(jnp.where(is_y, dy, 0.0))

    loss = (0.5 / N) * sum_x + (0.5 / N) * sum_y
    out_ref[...] = jnp.full((1, 1), loss, dtype=jnp.float32)


def kernel(inputs, targets):
    z = jnp.concatenate([inputs.reshape(-1), targets.reshape(-1)]).reshape(R, C)
    out = pl.pallas_call(
        _chamfer_body,
        out_shape=jax.ShapeDtypeStruct((1, 1), jnp.float32),
    )(z)
    return out[0, 0]
